# baseline (speedup 1.0000x reference)
import functools

import jax
import jax.numpy as jnp
from jax import lax
from jax.experimental import pallas as pl
from jax.experimental.pallas import tpu as pltpu
from jax.experimental.pallas import tpu_sc as plsc

N = 10000
D_IN = 128
D_E = 16
D_OUT = 128
E_EACH = 320000

NC = 2
NS = 16
NW = NC * NS
CHUNK = 128
CPW = 160
IDX_CPW = CPW + 1
E_PAD = NW * CPW * CHUNK
ROWS_PER_TILE = 632
N_PAD = NS * ROWS_PER_TILE

ROW_BLOCK = 1000


def _matmul_body(h_ref, e_ref, wh_ref, we_ref, z_ref):
    z_ref[...] = (
        jnp.dot(h_ref[...], wh_ref[...], preferred_element_type=jnp.float32)
        + jnp.dot(e_ref[...], we_ref[...], preferred_element_type=jnp.float32)
    )


def _add_body(a_ref, b_ref, o_ref):
    o_ref[...] = a_ref[...] + b_ref[...]


def _edge_body(z_hbm, src_hbm, dst_hbm, zrows_hbm, out_hbm,
               si, di, rows, acc):
    c = lax.axis_index("c")
    s = lax.axis_index("s")
    wid = s * NC + c
    row0 = s * ROWS_PER_TILE
    base = wid * IDX_CPW * CHUNK

    pltpu.sync_copy(zrows_hbm, acc.at[pl.ds(row0, ROWS_PER_TILE)])
    plsc.subcore_barrier()

    @pl.loop(0, CPW)
    def _chunk(j):
        off = base + j * CHUNK
        pltpu.sync_copy(src_hbm.at[pl.ds(off, CHUNK)], si)
        pltpu.sync_copy(z_hbm.at[si], rows)
        pltpu.sync_copy(dst_hbm.at[pl.ds(off, CHUNK)], di)
        pltpu.sync_copy(rows, acc.at[di], add=True)

    plsc.subcore_barrier()

    pltpu.sync_copy(
        acc.at[pl.ds(row0, ROWS_PER_TILE)],
        out_hbm.at[pl.ds(c * N_PAD + row0, ROWS_PER_TILE)],
    )


@jax.jit
def kernel(h_feat, e_feat, rain0, edge_index_xx, edge_index_yy, W_t):
    del rain0
    wh_t = W_t[:, :D_IN].T
    we_t = W_t[:, D_IN:].T
    n_blocks = N // ROW_BLOCK
    z = pl.pallas_call(
        _matmul_body,
        grid=(n_blocks,),
        in_specs=[
            pl.BlockSpec((ROW_BLOCK, D_IN), lambda i: (i, 0)),
            pl.BlockSpec((ROW_BLOCK, D_E), lambda i: (i, 0)),
            pl.BlockSpec((D_IN, D_OUT), lambda i: (0, 0)),
            pl.BlockSpec((D_E, D_OUT), lambda i: (0, 0)),
        ],
        out_specs=pl.BlockSpec((ROW_BLOCK, D_OUT), lambda i: (i, 0)),
        out_shape=jax.ShapeDtypeStruct((N, D_OUT), jnp.float32),
    )(h_feat, e_feat, wh_t, we_t)

    pad_grp = ((0, 0), (0, (IDX_CPW - CPW) * CHUNK))
    src = jnp.pad(jnp.concatenate(
        [edge_index_xx[0], edge_index_yy[0],
         jnp.zeros((E_PAD - 2 * E_EACH,), jnp.int32)]
    ).astype(jnp.int32).reshape(NW, CPW * CHUNK), pad_grp).reshape(-1)
    dst = jnp.pad(jnp.concatenate(
        [edge_index_xx[1], edge_index_yy[1],
         jnp.full((E_PAD - 2 * E_EACH,), N, jnp.int32)]
    ).astype(jnp.int32).reshape(NW, CPW * CHUNK), pad_grp).reshape(-1)
    zrows = jnp.zeros((ROWS_PER_TILE, D_OUT), jnp.float32)

    mesh = plsc.VectorSubcoreMesh(core_axis_name="c", subcore_axis_name="s")
    edge_kernel = functools.partial(
        pl.kernel,
        out_type=jax.ShapeDtypeStruct((NC * N_PAD, D_OUT), jnp.float32),
        mesh=mesh,
        scratch_types=[
            pltpu.VMEM((CHUNK,), jnp.int32),
            pltpu.VMEM((CHUNK,), jnp.int32),
            pltpu.VMEM((CHUNK, D_OUT), jnp.float32),
            pltpu.VMEM_SHARED((N_PAD, D_OUT), jnp.float32),
        ],
    )(_edge_body)
    partials = edge_kernel(z, src, dst, zrows)

    p0 = partials[:N]
    p1 = partials[N_PAD:N_PAD + N]
    h = pl.pallas_call(
        _add_body,
        grid=(n_blocks,),
        in_specs=[
            pl.BlockSpec((ROW_BLOCK, D_OUT), lambda i: (i, 0)),
            pl.BlockSpec((ROW_BLOCK, D_OUT), lambda i: (i, 0)),
        ],
        out_specs=pl.BlockSpec((ROW_BLOCK, D_OUT), lambda i: (i, 0)),
        out_shape=jax.ShapeDtypeStruct((N, D_OUT), jnp.float32),
    )(p0, p1)
    return h


# speedup vs baseline: 2.2471x; 2.2471x over previous
import functools

import jax
import jax.numpy as jnp
from jax import lax
from jax.experimental import pallas as pl
from jax.experimental.pallas import tpu as pltpu
from jax.experimental.pallas import tpu_sc as plsc

N = 10000
D_IN = 128
D_E = 16
D_OUT = 128
E_EACH = 320000

NC = 2
NS = 16
NW = NC * NS
CHUNK = 128
CPW = 157
IDX_CPW = CPW
E_PAD = NW * CPW * CHUNK
ROWS_PER_TILE = 632
N_PAD = NS * ROWS_PER_TILE

ROW_BLOCK = 1000


def _matmul_body(h_ref, e_ref, wh_ref, we_ref, z_ref):
    z_ref[...] = (
        jnp.dot(h_ref[...], wh_ref[...], preferred_element_type=jnp.float32)
        + jnp.dot(e_ref[...], we_ref[...], preferred_element_type=jnp.float32)
    )


def _add_body(a_ref, b_ref, o_ref):
    o_ref[...] = a_ref[...] + b_ref[...]


def _edge_body(z_hbm, src_hbm, dst_hbm, zrows_hbm, out_hbm,
               si, di, rows, acc):
    c = lax.axis_index("c")
    s = lax.axis_index("s")
    wid = s * NC + c
    row0 = s * ROWS_PER_TILE
    base = wid * IDX_CPW * CHUNK

    pltpu.sync_copy(zrows_hbm, acc.at[pl.ds(row0, ROWS_PER_TILE)])
    plsc.subcore_barrier()

    @pl.loop(0, CPW)
    def _chunk(j):
        off = base + j * CHUNK
        pltpu.sync_copy(src_hbm.at[pl.ds(off, CHUNK)], si)
        pltpu.sync_copy(z_hbm.at[si], rows)
        pltpu.sync_copy(dst_hbm.at[pl.ds(off, CHUNK)], di)
        pltpu.sync_copy(rows, acc.at[di], add=True)

    plsc.subcore_barrier()

    pltpu.sync_copy(
        acc.at[pl.ds(row0, ROWS_PER_TILE)],
        out_hbm.at[pl.ds(c * N_PAD + row0, ROWS_PER_TILE)],
    )


@jax.jit
def kernel(h_feat, e_feat, rain0, edge_index_xx, edge_index_yy, W_t):
    del rain0
    wh_t = W_t[:, :D_IN].T
    we_t = W_t[:, D_IN:].T
    n_blocks = N // ROW_BLOCK
    z = pl.pallas_call(
        _matmul_body,
        grid=(n_blocks,),
        in_specs=[
            pl.BlockSpec((ROW_BLOCK, D_IN), lambda i: (i, 0)),
            pl.BlockSpec((ROW_BLOCK, D_E), lambda i: (i, 0)),
            pl.BlockSpec((D_IN, D_OUT), lambda i: (0, 0)),
            pl.BlockSpec((D_E, D_OUT), lambda i: (0, 0)),
        ],
        out_specs=pl.BlockSpec((ROW_BLOCK, D_OUT), lambda i: (i, 0)),
        out_shape=jax.ShapeDtypeStruct((N, D_OUT), jnp.float32),
    )(h_feat, e_feat, wh_t, we_t)

    src = jnp.concatenate(
        [edge_index_xx[0], edge_index_yy[0],
         jnp.zeros((E_PAD - 2 * E_EACH,), jnp.int32)]).astype(jnp.int32)
    dst = jnp.concatenate(
        [edge_index_xx[1], edge_index_yy[1],
         jnp.full((E_PAD - 2 * E_EACH,), N, jnp.int32)]).astype(jnp.int32)
    zrows = jnp.zeros((ROWS_PER_TILE, D_OUT), jnp.float32)

    mesh = plsc.VectorSubcoreMesh(core_axis_name="c", subcore_axis_name="s")
    edge_kernel = functools.partial(
        pl.kernel,
        out_type=jax.ShapeDtypeStruct((NC * N_PAD, D_OUT), jnp.float32),
        mesh=mesh,
        scratch_types=[
            pltpu.VMEM((CHUNK,), jnp.int32),
            pltpu.VMEM((CHUNK,), jnp.int32),
            pltpu.VMEM((CHUNK, D_OUT), jnp.float32),
            pltpu.VMEM_SHARED((N_PAD, D_OUT), jnp.float32),
        ],
    )(_edge_body)
    partials = edge_kernel(z, src, dst, zrows)

    p0 = partials[:N]
    p1 = partials[N_PAD:N_PAD + N]
    h = pl.pallas_call(
        _add_body,
        grid=(n_blocks,),
        in_specs=[
            pl.BlockSpec((ROW_BLOCK, D_OUT), lambda i: (i, 0)),
            pl.BlockSpec((ROW_BLOCK, D_OUT), lambda i: (i, 0)),
        ],
        out_specs=pl.BlockSpec((ROW_BLOCK, D_OUT), lambda i: (i, 0)),
        out_shape=jax.ShapeDtypeStruct((N, D_OUT), jnp.float32),
    )(p0, p1)
    return h


# speedup vs baseline: 2.3839x; 1.0609x over previous
import functools

import jax
import jax.numpy as jnp
from jax import lax
from jax.experimental import pallas as pl
from jax.experimental.pallas import tpu as pltpu
from jax.experimental.pallas import tpu_sc as plsc

N = 10000
D_IN = 128
D_E = 16
D_OUT = 128
E_EACH = 320000

NC = 2
NS = 16
NW = NC * NS
CHUNK = 128
CPW = 157
IDX_CPW = CPW
E_PAD = NW * CPW * CHUNK
ROWS_PER_TILE = 632
N_PAD = NS * ROWS_PER_TILE

ROW_BLOCK = 1000


def _matmul_body(h_ref, e_ref, wh_ref, we_ref, z_ref):
    z_ref[...] = (
        jnp.dot(h_ref[...], wh_ref[...], preferred_element_type=jnp.float32)
        + jnp.dot(e_ref[...], we_ref[...], preferred_element_type=jnp.float32)
    )


def _add_body(a_ref, b_ref, o_ref):
    o_ref[...] = a_ref[...] + b_ref[...]


def _edge_body(z_hbm, ds_hbm, zrows_hbm, out_hbm, di, rows, acc):
    c = lax.axis_index("c")
    s = lax.axis_index("s")
    wid = s * NC + c
    row0 = s * ROWS_PER_TILE
    base = wid * IDX_CPW * (2 * CHUNK)

    pltpu.sync_copy(zrows_hbm, acc.at[pl.ds(row0, ROWS_PER_TILE)])
    plsc.subcore_barrier()

    @pl.loop(0, CPW)
    def _chunk(j):
        off = base + j * (2 * CHUNK)
        pltpu.sync_copy(ds_hbm.at[pl.ds(off, 2 * CHUNK)], di)
        pltpu.sync_copy(z_hbm.at[di.at[pl.ds(CHUNK, CHUNK)]], rows)
        pltpu.sync_copy(rows, acc.at[di.at[pl.ds(0, CHUNK)]], add=True)

    plsc.subcore_barrier()

    pltpu.sync_copy(
        acc.at[pl.ds(row0, ROWS_PER_TILE)],
        out_hbm.at[pl.ds(c * N_PAD + row0, ROWS_PER_TILE)],
    )


@jax.jit
def kernel(h_feat, e_feat, rain0, edge_index_xx, edge_index_yy, W_t):
    del rain0
    wh_t = W_t[:, :D_IN].T
    we_t = W_t[:, D_IN:].T
    n_blocks = N // ROW_BLOCK
    z = pl.pallas_call(
        _matmul_body,
        grid=(n_blocks,),
        in_specs=[
            pl.BlockSpec((ROW_BLOCK, D_IN), lambda i: (i, 0)),
            pl.BlockSpec((ROW_BLOCK, D_E), lambda i: (i, 0)),
            pl.BlockSpec((D_IN, D_OUT), lambda i: (0, 0)),
            pl.BlockSpec((D_E, D_OUT), lambda i: (0, 0)),
        ],
        out_specs=pl.BlockSpec((ROW_BLOCK, D_OUT), lambda i: (i, 0)),
        out_shape=jax.ShapeDtypeStruct((N, D_OUT), jnp.float32),
    )(h_feat, e_feat, wh_t, we_t)

    src = jnp.concatenate(
        [edge_index_xx[0], edge_index_yy[0],
         jnp.zeros((E_PAD - 2 * E_EACH,), jnp.int32)]).astype(jnp.int32)
    dst = jnp.concatenate(
        [edge_index_xx[1], edge_index_yy[1],
         jnp.full((E_PAD - 2 * E_EACH,), N, jnp.int32)]).astype(jnp.int32)
    ds_idx = jnp.concatenate(
        [dst.reshape(-1, 1, CHUNK), src.reshape(-1, 1, CHUNK)], axis=1
    ).reshape(-1)
    zrows = jnp.zeros((ROWS_PER_TILE, D_OUT), jnp.float32)

    mesh = plsc.VectorSubcoreMesh(core_axis_name="c", subcore_axis_name="s")
    edge_kernel = functools.partial(
        pl.kernel,
        out_type=jax.ShapeDtypeStruct((NC * N_PAD, D_OUT), jnp.float32),
        mesh=mesh,
        scratch_types=[
            pltpu.VMEM((2 * CHUNK,), jnp.int32),
            pltpu.VMEM((CHUNK, D_OUT), jnp.float32),
            pltpu.VMEM_SHARED((N_PAD, D_OUT), jnp.float32),
        ],
    )(_edge_body)
    partials = edge_kernel(z, ds_idx, zrows)

    p0 = partials[:N]
    p1 = partials[N_PAD:N_PAD + N]
    h = pl.pallas_call(
        _add_body,
        grid=(n_blocks,),
        in_specs=[
            pl.BlockSpec((ROW_BLOCK, D_OUT), lambda i: (i, 0)),
            pl.BlockSpec((ROW_BLOCK, D_OUT), lambda i: (i, 0)),
        ],
        out_specs=pl.BlockSpec((ROW_BLOCK, D_OUT), lambda i: (i, 0)),
        out_shape=jax.ShapeDtypeStruct((N, D_OUT), jnp.float32),
    )(p0, p1)
    return h
